# baseline (speedup 1.0000x reference)
import functools
import math

import jax
import jax.numpy as jnp
from jax import lax
from jax.experimental import pallas as pl
from jax.experimental.pallas import tpu as pltpu
from jax.experimental.pallas import tpu_sc as plsc

F32 = jnp.float32

T = 2048
DM = 1024
E = 16
DF = 4096
TOPK = 2
CAP = int(math.ceil(1.2 * T * TOPK / E))
CP = 320
ROWS = E * CP
DEAD = CAP

NW = 32
CH = 32
NCH = ROWS // NW // CH
TPW = T // NW
NC2 = TPW // CH


def _routing_body(x_ref, wg_ref, bg_ref, alpha_ref, inv_ref, slot_ref, gs_ref):
    x = x_ref[...]
    logits = lax.dot_general(
        x, wg_ref[...], (((1,), (0,)), ((), ())),
        precision=lax.Precision.DEFAULT,
        preferred_element_type=F32) + bg_ref[...]

    lane = lax.broadcasted_iota(jnp.int32, (T, E), 1)
    max1 = jnp.max(logits, axis=1, keepdims=True)
    idx1 = jnp.min(jnp.where(logits == max1, lane, E), axis=1, keepdims=True)
    masked = jnp.where(lane == idx1, -jnp.inf, logits)
    max2 = jnp.max(masked, axis=1, keepdims=True)
    idx2 = jnp.min(jnp.where(masked == max2, lane, E), axis=1, keepdims=True)

    z = jnp.exp(logits - max1)
    denom_sm = jnp.sum(z, axis=1, keepdims=True)
    p1 = jnp.sum(jnp.where(lane == idx1, z, 0.0), axis=1, keepdims=True) / denom_sm
    p2 = jnp.sum(jnp.where(lane == idx2, z, 0.0), axis=1, keepdims=True) / denom_sm
    gsum = p1 + p2 + 1e-9
    a1 = jnp.sum(jnp.where(lane == idx1, alpha_ref[...], 0.0), axis=1, keepdims=True)
    a2 = jnp.sum(jnp.where(lane == idx2, alpha_ref[...], 0.0), axis=1, keepdims=True)
    g1 = p1 / gsum * a1
    g2 = p2 / gsum * a2

    mA = (lane == idx1).astype(F32)
    mB = (lane == idx2).astype(F32)

    rr = lax.broadcasted_iota(jnp.int32, (T, T), 0)
    cc = lax.broadcasted_iota(jnp.int32, (T, T), 1)
    L = (rr >= cc).astype(F32)
    mAB = jnp.concatenate([mA, mB], axis=1)
    cAB = lax.dot_general(L, mAB, (((1,), (0,)), ((), ())),
                          precision=lax.Precision.DEFAULT,
                          preferred_element_type=F32)
    cA = cAB[:, :E]
    cB = cAB[:, E:]
    offs = cA[T - 1:T, :]
    locA = cA - 1.0
    locB = cB - 1.0 + offs
    posA = jnp.sum(jnp.where(mA > 0, locA, 0.0), axis=1, keepdims=True)
    posB = jnp.sum(jnp.where(mB > 0, locB, 0.0), axis=1, keepdims=True)
    vA = posA < float(CAP)
    vB = posB < float(CAP)

    posA_i = posA.astype(jnp.int32)
    posB_i = posB.astype(jnp.int32)
    slotA = jnp.where(vA, idx1 * CP + posA_i, DEAD)
    slotB = jnp.where(vB, idx2 * CP + posB_i, DEAD)
    slot_ref[...] = jnp.concatenate([slotA, slotB], axis=1)

    lane_cp = lax.broadcasted_iota(jnp.int32, (T, CP), 1)
    tcol = lax.broadcasted_iota(jnp.int32, (T, 1), 0).astype(F32)
    pohA = ((lane_cp == posA_i) & vA).astype(F32)
    pohB = ((lane_cp == posB_i) & vB).astype(F32)
    dn = (((0,), (0,)), ((), ()))
    invA = lax.dot_general(mA, pohA * tcol, dn,
                           precision=lax.Precision.HIGHEST,
                           preferred_element_type=F32)
    invB = lax.dot_general(mB, pohB * tcol, dn,
                           precision=lax.Precision.HIGHEST,
                           preferred_element_type=F32)
    inv_ref[...] = (invA + invB).astype(jnp.int32)
    wA = jnp.where(vA, g1, 0.0)
    wB = jnp.where(vB, g2, 0.0)
    gsA = lax.dot_general(mA, pohA * wA, dn,
                          precision=lax.Precision.HIGHEST,
                          preferred_element_type=F32)
    gsB = lax.dot_general(mB, pohB * wB, dn,
                          precision=lax.Precision.HIGHEST,
                          preferred_element_type=F32)
    gs_ref[...] = gsA + gsB


def _routing_call(x2d, Wg, bg2, alpha2):
    return pl.pallas_call(
        _routing_body,
        out_shape=(
            jax.ShapeDtypeStruct((E, CP), jnp.int32),
            jax.ShapeDtypeStruct((T, TOPK), jnp.int32),
            jax.ShapeDtypeStruct((E, CP), F32),
        ),
    )(x2d, Wg, bg2, alpha2)


def _dispatch_call(table, idx3, n_rows, nch, ch):
    mesh = plsc.VectorSubcoreMesh(core_axis_name="c", subcore_axis_name="s")
    rpw = nch * ch
    assert n_rows == NW * rpw

    @functools.partial(
        pl.kernel, mesh=mesh,
        out_type=jax.ShapeDtypeStruct((n_rows, DM), F32),
        scratch_types=[
            pltpu.VMEM((nch, ch), jnp.int32),
            pltpu.VMEM((2, ch, DM), F32),
            pltpu.SemaphoreType.DMA,
            pltpu.SemaphoreType.DMA,
            pltpu.SemaphoreType.DMA,
            pltpu.SemaphoreType.DMA,
        ],
    )
    def dispatch_kernel(table_hbm, idx_hbm, out_hbm, idx_v, rows_v,
                        gsem0, gsem1, wsem0, wsem1):
        wid = lax.axis_index("s") * 2 + lax.axis_index("c")
        base = wid * rpw
        gsems = (gsem0, gsem1)
        wsems = (wsem0, wsem1)
        pltpu.sync_copy(idx_hbm.at[wid], idx_v)
        gcp = {0: pltpu.async_copy(table_hbm.at[idx_v.at[0]], rows_v.at[0], gsems[0])}
        wcp = {}
        for k in range(nch):
            b = k % 2
            if k + 1 < nch:
                if k - 1 >= 0:
                    wcp[k - 1].wait()
                gcp[k + 1] = pltpu.async_copy(
                    table_hbm.at[idx_v.at[k + 1]], rows_v.at[1 - b], gsems[1 - b])
            gcp[k].wait()
            wcp[k] = pltpu.async_copy(
                rows_v.at[b], out_hbm.at[pl.ds(base + k * ch, ch)], wsems[b])
        if nch >= 2:
            wcp[nch - 2].wait()
        wcp[nch - 1].wait()

    return dispatch_kernel(table, idx3)


BTY = 512
BS = 512
NSB = ROWS // BS


def _combine_body(slot_ref, table_ref, y_ref):
    sb = pl.program_id(1)
    s0 = slot_ref[:, 0:1] - sb * BS
    s1 = slot_ref[:, 1:2] - sb * BS
    lane = lax.broadcasted_iota(jnp.int32, (BTY, BS), 1)
    oh = ((lane == s0) | (lane == s1)).astype(jnp.bfloat16)
    contrib = jnp.dot(oh, table_ref[...].astype(jnp.bfloat16),
                      preferred_element_type=F32)

    @pl.when(sb == 0)
    def _():
        y_ref[...] = contrib

    @pl.when(sb > 0)
    def _():
        y_ref[...] = y_ref[...] + contrib


def _combine_call(table, slots):
    return pl.pallas_call(
        _combine_body,
        grid=(T // BTY, NSB),
        in_specs=[
            pl.BlockSpec((BTY, TOPK), lambda ty, sb: (ty, 0)),
            pl.BlockSpec((BS, DM), lambda ty, sb: (sb, 0)),
        ],
        out_specs=pl.BlockSpec((BTY, DM), lambda ty, sb: (ty, 0)),
        out_shape=jax.ShapeDtypeStruct((T, DM), F32),
    )(slots, table)


BF = 512
NFB = DF // BF


def _gelu_new(x):
    return 0.5 * x * (1.0 + jnp.tanh(0.7978845608028654 * (x + 0.044715 * x * x * x)))


def _ffn_body(x_ref, w1_ref, b1_ref, w2_ref, b2_ref, gs_ref, out_ref):
    fb = pl.program_id(1)
    x = x_ref[0].astype(jnp.bfloat16)
    w1 = w1_ref[0].astype(jnp.bfloat16)
    h = jnp.dot(x, w1, preferred_element_type=F32) + b1_ref[0, 0]
    h = _gelu_new(h)
    w2 = w2_ref[0].astype(jnp.bfloat16)
    contrib = jnp.dot(h.astype(jnp.bfloat16), w2, preferred_element_type=F32)

    @pl.when(fb == 0)
    def _():
        out_ref[0] = contrib

    @pl.when(fb > 0)
    def _():
        out_ref[0] = out_ref[0] + contrib

    @pl.when(fb == NFB - 1)
    def _():
        out_ref[0] = (out_ref[0] + b2_ref[0]) * gs_ref[0]


def _ffn_call(xbuf_half, W1, b1, W2, b2, gslot, e_off, n_e):
    return pl.pallas_call(
        _ffn_body,
        grid=(n_e, NFB),
        in_specs=[
            pl.BlockSpec((1, CP, DM), lambda e, fb: (e, 0, 0)),
            pl.BlockSpec((1, DM, BF), lambda e, fb: (e + e_off, 0, fb)),
            pl.BlockSpec((1, 1, 1, BF), lambda e, fb: (e + e_off, fb, 0, 0)),
            pl.BlockSpec((1, BF, DM), lambda e, fb: (e + e_off, fb, 0)),
            pl.BlockSpec((1, 1, DM), lambda e, fb: (e + e_off, 0, 0)),
            pl.BlockSpec((1, CP, 1), lambda e, fb: (e + e_off, 0, 0)),
        ],
        out_specs=pl.BlockSpec((1, CP, DM), lambda e, fb: (e, 0, 0)),
        out_shape=jax.ShapeDtypeStruct((n_e, CP, DM), F32),
    )(xbuf_half, W1, b1.reshape(E, NFB, 1, BF), W2, b2.reshape(E, 1, DM),
      gslot.reshape(E, CP, 1))


def kernel(hidden_states, Wg, bg, W1, b1, W2, b2, alpha):
    b, s, d = hidden_states.shape
    assert b * s == T and d == DM and Wg.shape == (DM, E)

    x2d = hidden_states.reshape(T, DM)
    inv, slots, gslot = _routing_call(
        x2d, Wg, bg.reshape(1, E), alpha.reshape(1, E))

    xbuf = _dispatch_call(x2d, inv.reshape(NW, NCH, CH), ROWS, NCH, CH)
    out = _ffn_call(xbuf.reshape(E, CP, DM), W1, b1, W2, b2, gslot, 0, E)

    y = _combine_call(out.reshape(ROWS, DM), slots)
    return y.reshape(b, s, d)


# speedup vs baseline: 1.4969x; 1.4969x over previous
import functools
import math

import jax
import jax.numpy as jnp
from jax import lax
from jax.experimental import pallas as pl
from jax.experimental.pallas import tpu as pltpu
from jax.experimental.pallas import tpu_sc as plsc

F32 = jnp.float32

T = 2048
DM = 1024
E = 16
DF = 4096
TOPK = 2
CAP = int(math.ceil(1.2 * T * TOPK / E))
CP = 320
ROWS = E * CP
DEAD = CAP

NW = 32
CH = 32
NCH = ROWS // NW // CH
TPW = T // NW
NC2 = TPW // CH


def _routing_body(x_ref, wg_ref, bg_ref, alpha_ref, inv_ref, slot_ref, gs_ref):
    x = x_ref[...]
    logits = lax.dot_general(
        x, wg_ref[...], (((1,), (0,)), ((), ())),
        precision=lax.Precision.DEFAULT,
        preferred_element_type=F32) + bg_ref[...]

    lane = lax.broadcasted_iota(jnp.int32, (T, E), 1)
    max1 = jnp.max(logits, axis=1, keepdims=True)
    idx1 = jnp.min(jnp.where(logits == max1, lane, E), axis=1, keepdims=True)
    masked = jnp.where(lane == idx1, -jnp.inf, logits)
    max2 = jnp.max(masked, axis=1, keepdims=True)
    idx2 = jnp.min(jnp.where(masked == max2, lane, E), axis=1, keepdims=True)

    z = jnp.exp(logits - max1)
    denom_sm = jnp.sum(z, axis=1, keepdims=True)
    p1 = jnp.sum(jnp.where(lane == idx1, z, 0.0), axis=1, keepdims=True) / denom_sm
    p2 = jnp.sum(jnp.where(lane == idx2, z, 0.0), axis=1, keepdims=True) / denom_sm
    gsum = p1 + p2 + 1e-9
    a1 = jnp.sum(jnp.where(lane == idx1, alpha_ref[...], 0.0), axis=1, keepdims=True)
    a2 = jnp.sum(jnp.where(lane == idx2, alpha_ref[...], 0.0), axis=1, keepdims=True)
    g1 = p1 / gsum * a1
    g2 = p2 / gsum * a2

    mA = (lane == idx1).astype(F32)
    mB = (lane == idx2).astype(F32)

    rr = lax.broadcasted_iota(jnp.int32, (T, T), 0)
    cc = lax.broadcasted_iota(jnp.int32, (T, T), 1)
    L = (rr >= cc).astype(F32)
    mAB = jnp.concatenate([mA, mB], axis=1)
    cAB = lax.dot_general(L, mAB, (((1,), (0,)), ((), ())),
                          precision=lax.Precision.DEFAULT,
                          preferred_element_type=F32)
    cA = cAB[:, :E]
    cB = cAB[:, E:]
    offs = cA[T - 1:T, :]
    locA = cA - 1.0
    locB = cB - 1.0 + offs
    posA = jnp.sum(jnp.where(mA > 0, locA, 0.0), axis=1, keepdims=True)
    posB = jnp.sum(jnp.where(mB > 0, locB, 0.0), axis=1, keepdims=True)
    vA = posA < float(CAP)
    vB = posB < float(CAP)

    posA_i = posA.astype(jnp.int32)
    posB_i = posB.astype(jnp.int32)
    slotA = jnp.where(vA, idx1 * CP + posA_i, DEAD)
    slotB = jnp.where(vB, idx2 * CP + posB_i, DEAD)
    slot_ref[...] = jnp.concatenate([slotA, slotB], axis=1)

    lane_cp = lax.broadcasted_iota(jnp.int32, (T, CP), 1)
    tcol = lax.broadcasted_iota(jnp.int32, (T, 1), 0).astype(F32)
    pohA = ((lane_cp == posA_i) & vA).astype(F32)
    pohB = ((lane_cp == posB_i) & vB).astype(F32)
    dn = (((0,), (0,)), ((), ()))
    invA = lax.dot_general(mA, pohA * tcol, dn,
                           precision=lax.Precision.HIGHEST,
                           preferred_element_type=F32)
    invB = lax.dot_general(mB, pohB * tcol, dn,
                           precision=lax.Precision.HIGHEST,
                           preferred_element_type=F32)
    inv_ref[...] = (invA + invB).astype(jnp.int32)
    wA = jnp.where(vA, g1, 0.0)
    wB = jnp.where(vB, g2, 0.0)
    gsA = lax.dot_general(mA, pohA * wA, dn,
                          precision=lax.Precision.HIGHEST,
                          preferred_element_type=F32)
    gsB = lax.dot_general(mB, pohB * wB, dn,
                          precision=lax.Precision.HIGHEST,
                          preferred_element_type=F32)
    gs_ref[...] = gsA + gsB


def _routing_call(x2d, Wg, bg2, alpha2):
    return pl.pallas_call(
        _routing_body,
        out_shape=(
            jax.ShapeDtypeStruct((E, CP), jnp.int32),
            jax.ShapeDtypeStruct((T, TOPK), jnp.int32),
            jax.ShapeDtypeStruct((E, CP), F32),
        ),
    )(x2d, Wg, bg2, alpha2)


def _dispatch_call(table, idx3, n_rows, nch, ch):
    mesh = plsc.VectorSubcoreMesh(core_axis_name="c", subcore_axis_name="s")
    rpw = nch * ch
    assert n_rows == NW * rpw

    @functools.partial(
        pl.kernel, mesh=mesh,
        out_type=jax.ShapeDtypeStruct((n_rows, DM), F32),
        scratch_types=[
            pltpu.VMEM((nch, ch), jnp.int32),
            pltpu.VMEM((2, ch, DM), F32),
            pltpu.SemaphoreType.DMA,
            pltpu.SemaphoreType.DMA,
            pltpu.SemaphoreType.DMA,
            pltpu.SemaphoreType.DMA,
        ],
    )
    def dispatch_kernel(table_hbm, idx_hbm, out_hbm, idx_v, rows_v,
                        gsem0, gsem1, wsem0, wsem1):
        wid = lax.axis_index("s") * 2 + lax.axis_index("c")
        base = wid * rpw
        gsems = (gsem0, gsem1)
        wsems = (wsem0, wsem1)
        pltpu.sync_copy(idx_hbm.at[wid], idx_v)
        gcp = {0: pltpu.async_copy(table_hbm.at[idx_v.at[0]], rows_v.at[0], gsems[0])}
        wcp = {}
        for k in range(nch):
            b = k % 2
            if k + 1 < nch:
                if k - 1 >= 0:
                    wcp[k - 1].wait()
                gcp[k + 1] = pltpu.async_copy(
                    table_hbm.at[idx_v.at[k + 1]], rows_v.at[1 - b], gsems[1 - b])
            gcp[k].wait()
            wcp[k] = pltpu.async_copy(
                rows_v.at[b], out_hbm.at[pl.ds(base + k * ch, ch)], wsems[b])
        if nch >= 2:
            wcp[nch - 2].wait()
        wcp[nch - 1].wait()

    return dispatch_kernel(table, idx3)


def _combine_call(table, s0, s1):
    mesh = plsc.VectorSubcoreMesh(core_axis_name="c", subcore_axis_name="s")

    @functools.partial(
        pl.kernel, mesh=mesh,
        out_type=jax.ShapeDtypeStruct((T, DM), F32),
        scratch_types=[
            pltpu.VMEM((NC2, CH), jnp.int32),
            pltpu.VMEM((NC2, CH), jnp.int32),
            pltpu.VMEM((CH, DM), F32),
            pltpu.VMEM((CH, DM), F32),
            pltpu.SemaphoreType.DMA,
            pltpu.SemaphoreType.DMA,
        ],
    )
    def combine_kernel(table_hbm, s0_hbm, s1_hbm, y_hbm, i0_v, i1_v,
                       r0_v, r1_v, sem0, sem1):
        wid = lax.axis_index("s") * 2 + lax.axis_index("c")
        base = wid * TPW
        pltpu.sync_copy(s0_hbm.at[wid], i0_v)
        pltpu.sync_copy(s1_hbm.at[wid], i1_v)
        for k in range(NC2):
            c0 = pltpu.async_copy(table_hbm.at[i0_v.at[k]], r0_v, sem0)
            c1 = pltpu.async_copy(table_hbm.at[i1_v.at[k]], r1_v, sem1)
            c0.wait()
            c1.wait()

            def add_row(i, carry):
                for j in range(DM // 16):
                    sl = pl.ds(j * 16, 16)
                    r0_v[i, sl] = r0_v[i, sl] + r1_v[i, sl]
                return carry

            lax.fori_loop(0, CH, add_row, 0)
            pltpu.sync_copy(r0_v, y_hbm.at[pl.ds(base + k * CH, CH)])

    return combine_kernel(table, s0, s1)


BF = 2048
NFB = DF // BF


def _gelu_new(x):
    return 0.5 * x * (1.0 + jnp.tanh(0.7978845608028654 * (x + 0.044715 * x * x * x)))


def _ffn_body(x_ref, inv_ref, w1_ref, b1_ref, w2_ref, b2_ref, gs_ref,
              out_ref, xb_scr):
    fb = pl.program_id(1)

    @pl.when(fb == 0)
    def _():
        tl = lax.broadcasted_iota(jnp.int32, (CP, T), 1)
        D = (tl == inv_ref[0]).astype(F32)
        xb_scr[...] = jnp.dot(D, x_ref[...], precision=lax.Precision.DEFAULT,
                              preferred_element_type=F32)

    h = jnp.dot(xb_scr[...], w1_ref[0], precision=lax.Precision.DEFAULT,
                preferred_element_type=F32) + b1_ref[0, 0]
    h = _gelu_new(h)
    contrib = jnp.dot(h, w2_ref[0], precision=lax.Precision.DEFAULT,
                      preferred_element_type=F32)

    @pl.when(fb == 0)
    def _():
        out_ref[0] = contrib

    @pl.when(fb > 0)
    def _():
        out_ref[0] = out_ref[0] + contrib

    @pl.when(fb == NFB - 1)
    def _():
        out_ref[0] = (out_ref[0] + b2_ref[0]) * gs_ref[0]


def _ffn_call(x2d, inv, W1, b1, W2, b2, gslot):
    return pl.pallas_call(
        _ffn_body,
        grid=(E, NFB),
        in_specs=[
            pl.BlockSpec((T, DM), lambda e, fb: (0, 0)),
            pl.BlockSpec((1, CP, 1), lambda e, fb: (e, 0, 0)),
            pl.BlockSpec((1, DM, BF), lambda e, fb: (e, 0, fb)),
            pl.BlockSpec((1, 1, 1, BF), lambda e, fb: (e, fb, 0, 0)),
            pl.BlockSpec((1, BF, DM), lambda e, fb: (e, fb, 0)),
            pl.BlockSpec((1, 1, DM), lambda e, fb: (e, 0, 0)),
            pl.BlockSpec((1, CP, 1), lambda e, fb: (e, 0, 0)),
        ],
        out_specs=pl.BlockSpec((1, CP, DM), lambda e, fb: (e, 0, 0)),
        out_shape=jax.ShapeDtypeStruct((E, CP, DM), F32),
        scratch_shapes=[pltpu.VMEM((CP, DM), F32)],
    )(x2d, inv.reshape(E, CP, 1), W1, b1.reshape(E, NFB, 1, BF), W2,
      b2.reshape(E, 1, DM), gslot.reshape(E, CP, 1))


def kernel(hidden_states, Wg, bg, W1, b1, W2, b2, alpha):
    b, s, d = hidden_states.shape
    assert b * s == T and d == DM and Wg.shape == (DM, E)

    x2d = hidden_states.reshape(T, DM)
    inv, slots, gslot = _routing_call(
        x2d, Wg, bg.reshape(1, E), alpha.reshape(1, E))

    out = _ffn_call(x2d, inv, W1, b1, W2, b2, gslot)

    s0 = slots[:, 0].reshape(NW, NC2, CH)
    s1 = slots[:, 1].reshape(NW, NC2, CH)
    y = _combine_call(out.reshape(ROWS, DM), s0, s1)
    return y.reshape(b, s, d)


# speedup vs baseline: 1.5023x; 1.0036x over previous
import functools
import math

import jax
import jax.numpy as jnp
from jax import lax
from jax.experimental import pallas as pl
from jax.experimental.pallas import tpu as pltpu
from jax.experimental.pallas import tpu_sc as plsc

F32 = jnp.float32

T = 2048
DM = 1024
E = 16
DF = 4096
TOPK = 2
CAP = int(math.ceil(1.2 * T * TOPK / E))
CP = 320
ROWS = E * CP
DEAD = CAP

NW = 32
CH = 32
NCH = ROWS // NW // CH
TPW = T // NW
CC = 16
NC2 = TPW // CC


def _routing_body(x_ref, wg_ref, bg_ref, alpha_ref, inv_ref, slot_ref, gs_ref):
    x = x_ref[...]
    logits = lax.dot_general(
        x, wg_ref[...], (((1,), (0,)), ((), ())),
        precision=lax.Precision.DEFAULT,
        preferred_element_type=F32) + bg_ref[...]

    lane = lax.broadcasted_iota(jnp.int32, (T, E), 1)
    max1 = jnp.max(logits, axis=1, keepdims=True)
    idx1 = jnp.min(jnp.where(logits == max1, lane, E), axis=1, keepdims=True)
    masked = jnp.where(lane == idx1, -jnp.inf, logits)
    max2 = jnp.max(masked, axis=1, keepdims=True)
    idx2 = jnp.min(jnp.where(masked == max2, lane, E), axis=1, keepdims=True)

    z = jnp.exp(logits - max1)
    denom_sm = jnp.sum(z, axis=1, keepdims=True)
    p1 = jnp.sum(jnp.where(lane == idx1, z, 0.0), axis=1, keepdims=True) / denom_sm
    p2 = jnp.sum(jnp.where(lane == idx2, z, 0.0), axis=1, keepdims=True) / denom_sm
    gsum = p1 + p2 + 1e-9
    a1 = jnp.sum(jnp.where(lane == idx1, alpha_ref[...], 0.0), axis=1, keepdims=True)
    a2 = jnp.sum(jnp.where(lane == idx2, alpha_ref[...], 0.0), axis=1, keepdims=True)
    g1 = p1 / gsum * a1
    g2 = p2 / gsum * a2

    mA = (lane == idx1).astype(F32)
    mB = (lane == idx2).astype(F32)

    rr = lax.broadcasted_iota(jnp.int32, (T, T), 0)
    cc = lax.broadcasted_iota(jnp.int32, (T, T), 1)
    L = (rr >= cc).astype(F32)
    mAB = jnp.concatenate([mA, mB], axis=1)
    cAB = lax.dot_general(L, mAB, (((1,), (0,)), ((), ())),
                          precision=lax.Precision.DEFAULT,
                          preferred_element_type=F32)
    cA = cAB[:, :E]
    cB = cAB[:, E:]
    offs = cA[T - 1:T, :]
    locA = cA - 1.0
    locB = cB - 1.0 + offs
    posA = jnp.sum(jnp.where(mA > 0, locA, 0.0), axis=1, keepdims=True)
    posB = jnp.sum(jnp.where(mB > 0, locB, 0.0), axis=1, keepdims=True)
    vA = posA < float(CAP)
    vB = posB < float(CAP)

    posA_i = posA.astype(jnp.int32)
    posB_i = posB.astype(jnp.int32)
    slotA = jnp.where(vA, idx1 * CP + posA_i, DEAD)
    slotB = jnp.where(vB, idx2 * CP + posB_i, DEAD)
    slot_ref[...] = jnp.concatenate([slotA, slotB], axis=1)

    lane_cp = lax.broadcasted_iota(jnp.int32, (T, CP), 1)
    tcol = lax.broadcasted_iota(jnp.int32, (T, 1), 0).astype(F32)
    pohA = ((lane_cp == posA_i) & vA).astype(F32)
    pohB = ((lane_cp == posB_i) & vB).astype(F32)
    dn = (((0,), (0,)), ((), ()))
    invA = lax.dot_general(mA, pohA * tcol, dn,
                           precision=lax.Precision.HIGHEST,
                           preferred_element_type=F32)
    invB = lax.dot_general(mB, pohB * tcol, dn,
                           precision=lax.Precision.HIGHEST,
                           preferred_element_type=F32)
    inv_ref[...] = (invA + invB).astype(jnp.int32)
    wA = jnp.where(vA, g1, 0.0)
    wB = jnp.where(vB, g2, 0.0)
    gsA = lax.dot_general(mA, pohA * wA, dn,
                          precision=lax.Precision.HIGHEST,
                          preferred_element_type=F32)
    gsB = lax.dot_general(mB, pohB * wB, dn,
                          precision=lax.Precision.HIGHEST,
                          preferred_element_type=F32)
    gs_ref[...] = gsA + gsB


def _routing_call(x2d, Wg, bg2, alpha2):
    return pl.pallas_call(
        _routing_body,
        out_shape=(
            jax.ShapeDtypeStruct((E, CP), jnp.int32),
            jax.ShapeDtypeStruct((T, TOPK), jnp.int32),
            jax.ShapeDtypeStruct((E, CP), F32),
        ),
    )(x2d, Wg, bg2, alpha2)


def _dispatch_call(table, idx3, n_rows, nch, ch):
    mesh = plsc.VectorSubcoreMesh(core_axis_name="c", subcore_axis_name="s")
    rpw = nch * ch
    assert n_rows == NW * rpw

    @functools.partial(
        pl.kernel, mesh=mesh,
        out_type=jax.ShapeDtypeStruct((n_rows, DM), F32),
        scratch_types=[
            pltpu.VMEM((nch, ch), jnp.int32),
            pltpu.VMEM((2, ch, DM), F32),
            pltpu.SemaphoreType.DMA,
            pltpu.SemaphoreType.DMA,
            pltpu.SemaphoreType.DMA,
            pltpu.SemaphoreType.DMA,
        ],
    )
    def dispatch_kernel(table_hbm, idx_hbm, out_hbm, idx_v, rows_v,
                        gsem0, gsem1, wsem0, wsem1):
        wid = lax.axis_index("s") * 2 + lax.axis_index("c")
        base = wid * rpw
        gsems = (gsem0, gsem1)
        wsems = (wsem0, wsem1)
        pltpu.sync_copy(idx_hbm.at[wid], idx_v)
        gcp = {0: pltpu.async_copy(table_hbm.at[idx_v.at[0]], rows_v.at[0], gsems[0])}
        wcp = {}
        for k in range(nch):
            b = k % 2
            if k + 1 < nch:
                if k - 1 >= 0:
                    wcp[k - 1].wait()
                gcp[k + 1] = pltpu.async_copy(
                    table_hbm.at[idx_v.at[k + 1]], rows_v.at[1 - b], gsems[1 - b])
            gcp[k].wait()
            wcp[k] = pltpu.async_copy(
                rows_v.at[b], out_hbm.at[pl.ds(base + k * ch, ch)], wsems[b])
        if nch >= 2:
            wcp[nch - 2].wait()
        wcp[nch - 1].wait()

    return dispatch_kernel(table, idx3)


def _combine_call(table, s0, s1):
    mesh = plsc.VectorSubcoreMesh(core_axis_name="c", subcore_axis_name="s")

    @functools.partial(
        pl.kernel, mesh=mesh,
        out_type=jax.ShapeDtypeStruct((T, DM), F32),
        scratch_types=[
            pltpu.VMEM((NC2, CC), jnp.int32),
            pltpu.VMEM((NC2, CC), jnp.int32),
            pltpu.VMEM((2, CC, DM), F32),
            pltpu.VMEM((2, CC, DM), F32),
            pltpu.SemaphoreType.DMA,
            pltpu.SemaphoreType.DMA,
        ],
    )
    def combine_kernel(table_hbm, s0_hbm, s1_hbm, y_hbm, i0_v, i1_v,
                       r0_v, r1_v, sem0, sem1):
        wid = lax.axis_index("s") * 2 + lax.axis_index("c")
        base = wid * TPW
        sems = (sem0, sem1)
        pltpu.sync_copy(s0_hbm.at[wid], i0_v)
        pltpu.sync_copy(s1_hbm.at[wid], i1_v)
        cp = {0: (pltpu.async_copy(table_hbm.at[i0_v.at[0]], r0_v.at[0], sems[0]),
                  pltpu.async_copy(table_hbm.at[i1_v.at[0]], r1_v.at[0], sems[0]))}
        for k in range(NC2):
            bb = k % 2
            if k + 1 < NC2:
                cp[k + 1] = (
                    pltpu.async_copy(table_hbm.at[i0_v.at[k + 1]],
                                     r0_v.at[1 - bb], sems[1 - bb]),
                    pltpu.async_copy(table_hbm.at[i1_v.at[k + 1]],
                                     r1_v.at[1 - bb], sems[1 - bb]))
            cp[k][0].wait()
            cp[k][1].wait()

            def add_row(i, carry, bb=bb):
                for j in range(DM // 16):
                    sl = pl.ds(j * 16, 16)
                    r0_v[bb, i, sl] = r0_v[bb, i, sl] + r1_v[bb, i, sl]
                return carry

            lax.fori_loop(0, CC, add_row, 0)
            pltpu.sync_copy(r0_v.at[bb], y_hbm.at[pl.ds(base + k * CC, CC)])

    return combine_kernel(table, s0, s1)


BF = 2048
NFB = DF // BF


def _gelu_new(x):
    return 0.5 * x * (1.0 + jnp.tanh(0.7978845608028654 * (x + 0.044715 * x * x * x)))


def _ffn_body(x_ref, inv_ref, w1_ref, b1_ref, w2_ref, b2_ref, gs_ref,
              out_ref, xb_scr):
    fb = pl.program_id(1)

    @pl.when(fb == 0)
    def _():
        tl = lax.broadcasted_iota(jnp.int32, (CP, T), 1)
        D = (tl == inv_ref[0]).astype(F32)
        xb_scr[...] = jnp.dot(D, x_ref[...], precision=lax.Precision.DEFAULT,
                              preferred_element_type=F32)

    h = jnp.dot(xb_scr[...], w1_ref[0], precision=lax.Precision.DEFAULT,
                preferred_element_type=F32) + b1_ref[0, 0]
    h = _gelu_new(h)
    contrib = jnp.dot(h, w2_ref[0], precision=lax.Precision.DEFAULT,
                      preferred_element_type=F32)

    @pl.when(fb == 0)
    def _():
        out_ref[0] = contrib

    @pl.when(fb > 0)
    def _():
        out_ref[0] = out_ref[0] + contrib

    @pl.when(fb == NFB - 1)
    def _():
        out_ref[0] = (out_ref[0] + b2_ref[0]) * gs_ref[0]


def _ffn_call(x2d, inv, W1, b1, W2, b2, gslot):
    return pl.pallas_call(
        _ffn_body,
        grid=(E, NFB),
        in_specs=[
            pl.BlockSpec((T, DM), lambda e, fb: (0, 0)),
            pl.BlockSpec((1, CP, 1), lambda e, fb: (e, 0, 0)),
            pl.BlockSpec((1, DM, BF), lambda e, fb: (e, 0, fb)),
            pl.BlockSpec((1, 1, 1, BF), lambda e, fb: (e, fb, 0, 0)),
            pl.BlockSpec((1, BF, DM), lambda e, fb: (e, fb, 0)),
            pl.BlockSpec((1, 1, DM), lambda e, fb: (e, 0, 0)),
            pl.BlockSpec((1, CP, 1), lambda e, fb: (e, 0, 0)),
        ],
        out_specs=pl.BlockSpec((1, CP, DM), lambda e, fb: (e, 0, 0)),
        out_shape=jax.ShapeDtypeStruct((E, CP, DM), F32),
        scratch_shapes=[pltpu.VMEM((CP, DM), F32)],
    )(x2d, inv.reshape(E, CP, 1), W1, b1.reshape(E, NFB, 1, BF), W2,
      b2.reshape(E, 1, DM), gslot.reshape(E, CP, 1))


def kernel(hidden_states, Wg, bg, W1, b1, W2, b2, alpha):
    b, s, d = hidden_states.shape
    assert b * s == T and d == DM and Wg.shape == (DM, E)

    x2d = hidden_states.reshape(T, DM)
    inv, slots, gslot = _routing_call(
        x2d, Wg, bg.reshape(1, E), alpha.reshape(1, E))

    out = _ffn_call(x2d, inv, W1, b1, W2, b2, gslot)

    s0 = slots[:, 0].reshape(NW, NC2, CC)
    s1 = slots[:, 1].reshape(NW, NC2, CC)
    y = _combine_call(out.reshape(ROWS, DM), s0, s1)
    return y.reshape(b, s, d)


# speedup vs baseline: 1.5264x; 1.0161x over previous
import functools
import math

import jax
import jax.numpy as jnp
from jax import lax
from jax.experimental import pallas as pl
from jax.experimental.pallas import tpu as pltpu
from jax.experimental.pallas import tpu_sc as plsc

F32 = jnp.float32

T = 2048
DM = 1024
E = 16
DF = 4096
TOPK = 2
CAP = int(math.ceil(1.2 * T * TOPK / E))
CP = 320
ROWS = E * CP
DEAD = CAP

NW = 32
CH = 32
NCH = ROWS // NW // CH
TPW = T // NW
CC = 16
NC2 = TPW // CC


def _routing_body(x_ref, wg_ref, bg_ref, alpha_ref, inv_ref, slot_ref, gs_ref):
    x = x_ref[...]
    logits = lax.dot_general(
        x, wg_ref[...], (((1,), (0,)), ((), ())),
        precision=lax.Precision.DEFAULT,
        preferred_element_type=F32) + bg_ref[...]

    lane = lax.broadcasted_iota(jnp.int32, (T, E), 1)
    max1 = jnp.max(logits, axis=1, keepdims=True)
    idx1 = jnp.min(jnp.where(logits == max1, lane, E), axis=1, keepdims=True)
    masked = jnp.where(lane == idx1, -jnp.inf, logits)
    max2 = jnp.max(masked, axis=1, keepdims=True)
    idx2 = jnp.min(jnp.where(masked == max2, lane, E), axis=1, keepdims=True)

    z = jnp.exp(logits - max1)
    denom_sm = jnp.sum(z, axis=1, keepdims=True)
    p1 = jnp.sum(jnp.where(lane == idx1, z, 0.0), axis=1, keepdims=True) / denom_sm
    p2 = jnp.sum(jnp.where(lane == idx2, z, 0.0), axis=1, keepdims=True) / denom_sm
    gsum = p1 + p2 + 1e-9
    a1 = jnp.sum(jnp.where(lane == idx1, alpha_ref[...], 0.0), axis=1, keepdims=True)
    a2 = jnp.sum(jnp.where(lane == idx2, alpha_ref[...], 0.0), axis=1, keepdims=True)
    g1 = p1 / gsum * a1
    g2 = p2 / gsum * a2

    mA = (lane == idx1).astype(F32)
    mB = (lane == idx2).astype(F32)

    rr = lax.broadcasted_iota(jnp.int32, (T, T), 0)
    cc = lax.broadcasted_iota(jnp.int32, (T, T), 1)
    L = (rr >= cc).astype(F32)
    mAB = jnp.concatenate([mA, mB], axis=1)
    cAB = lax.dot_general(L, mAB, (((1,), (0,)), ((), ())),
                          precision=lax.Precision.DEFAULT,
                          preferred_element_type=F32)
    cA = cAB[:, :E]
    cB = cAB[:, E:]
    offs = cA[T - 1:T, :]
    locA = cA - 1.0
    locB = cB - 1.0 + offs
    posA = jnp.sum(jnp.where(mA > 0, locA, 0.0), axis=1, keepdims=True)
    posB = jnp.sum(jnp.where(mB > 0, locB, 0.0), axis=1, keepdims=True)
    vA = posA < float(CAP)
    vB = posB < float(CAP)

    posA_i = posA.astype(jnp.int32)
    posB_i = posB.astype(jnp.int32)
    slotA = jnp.where(vA, idx1 * CP + posA_i, DEAD)
    slotB = jnp.where(vB, idx2 * CP + posB_i, DEAD)
    slot_ref[...] = jnp.concatenate([slotA, slotB], axis=1)

    lane_cp = lax.broadcasted_iota(jnp.int32, (T, CP), 1)
    tcol = lax.broadcasted_iota(jnp.int32, (T, 1), 0).astype(F32)
    thi = jnp.floor(tcol * (1.0 / 128.0))
    tlo = tcol - 128.0 * thi
    pohA = ((lane_cp == posA_i) & vA).astype(F32)
    pohB = ((lane_cp == posB_i) & vB).astype(F32)
    wA = jnp.where(vA, g1, 0.0)
    wB = jnp.where(vB, g2, 0.0)
    dn = (((0,), (0,)), ((), ()))
    rhsA = jnp.concatenate([pohA * thi, pohA * tlo, pohA * wA], axis=1)
    rhsB = jnp.concatenate([pohB * thi, pohB * tlo, pohB * wB], axis=1)
    resA = lax.dot_general(mA, rhsA, dn,
                           precision=lax.Precision.DEFAULT,
                           preferred_element_type=F32)
    resB = lax.dot_general(mB, rhsB, dn,
                           precision=lax.Precision.DEFAULT,
                           preferred_element_type=F32)
    res = resA + resB
    inv_ref[...] = (128.0 * res[:, :CP] + res[:, CP:2 * CP]).astype(jnp.int32)
    gs_ref[...] = res[:, 2 * CP:]


def _routing_call(x2d, Wg, bg2, alpha2):
    return pl.pallas_call(
        _routing_body,
        out_shape=(
            jax.ShapeDtypeStruct((E, CP), jnp.int32),
            jax.ShapeDtypeStruct((T, TOPK), jnp.int32),
            jax.ShapeDtypeStruct((E, CP), F32),
        ),
    )(x2d, Wg, bg2, alpha2)


def _dispatch_call(table, idx3, n_rows, nch, ch):
    mesh = plsc.VectorSubcoreMesh(core_axis_name="c", subcore_axis_name="s")
    rpw = nch * ch
    assert n_rows == NW * rpw

    @functools.partial(
        pl.kernel, mesh=mesh,
        out_type=jax.ShapeDtypeStruct((n_rows, DM), F32),
        scratch_types=[
            pltpu.VMEM((nch, ch), jnp.int32),
            pltpu.VMEM((2, ch, DM), F32),
            pltpu.SemaphoreType.DMA,
            pltpu.SemaphoreType.DMA,
            pltpu.SemaphoreType.DMA,
            pltpu.SemaphoreType.DMA,
        ],
    )
    def dispatch_kernel(table_hbm, idx_hbm, out_hbm, idx_v, rows_v,
                        gsem0, gsem1, wsem0, wsem1):
        wid = lax.axis_index("s") * 2 + lax.axis_index("c")
        base = wid * rpw
        gsems = (gsem0, gsem1)
        wsems = (wsem0, wsem1)
        pltpu.sync_copy(idx_hbm.at[wid], idx_v)
        gcp = {0: pltpu.async_copy(table_hbm.at[idx_v.at[0]], rows_v.at[0], gsems[0])}
        wcp = {}
        for k in range(nch):
            b = k % 2
            if k + 1 < nch:
                if k - 1 >= 0:
                    wcp[k - 1].wait()
                gcp[k + 1] = pltpu.async_copy(
                    table_hbm.at[idx_v.at[k + 1]], rows_v.at[1 - b], gsems[1 - b])
            gcp[k].wait()
            wcp[k] = pltpu.async_copy(
                rows_v.at[b], out_hbm.at[pl.ds(base + k * ch, ch)], wsems[b])
        if nch >= 2:
            wcp[nch - 2].wait()
        wcp[nch - 1].wait()

    return dispatch_kernel(table, idx3)


def _combine_call(table, s0, s1):
    mesh = plsc.VectorSubcoreMesh(core_axis_name="c", subcore_axis_name="s")

    @functools.partial(
        pl.kernel, mesh=mesh,
        out_type=jax.ShapeDtypeStruct((T, DM), F32),
        scratch_types=[
            pltpu.VMEM((NC2, CC), jnp.int32),
            pltpu.VMEM((NC2, CC), jnp.int32),
            pltpu.VMEM((2, CC, DM), F32),
            pltpu.VMEM((2, CC, DM), F32),
            pltpu.SemaphoreType.DMA,
            pltpu.SemaphoreType.DMA,
        ],
    )
    def combine_kernel(table_hbm, s0_hbm, s1_hbm, y_hbm, i0_v, i1_v,
                       r0_v, r1_v, sem0, sem1):
        wid = lax.axis_index("s") * 2 + lax.axis_index("c")
        base = wid * TPW
        sems = (sem0, sem1)
        pltpu.sync_copy(s0_hbm.at[wid], i0_v)
        pltpu.sync_copy(s1_hbm.at[wid], i1_v)
        cp = {0: (pltpu.async_copy(table_hbm.at[i0_v.at[0]], r0_v.at[0], sems[0]),
                  pltpu.async_copy(table_hbm.at[i1_v.at[0]], r1_v.at[0], sems[0]))}
        for k in range(NC2):
            bb = k % 2
            if k + 1 < NC2:
                cp[k + 1] = (
                    pltpu.async_copy(table_hbm.at[i0_v.at[k + 1]],
                                     r0_v.at[1 - bb], sems[1 - bb]),
                    pltpu.async_copy(table_hbm.at[i1_v.at[k + 1]],
                                     r1_v.at[1 - bb], sems[1 - bb]))
            cp[k][0].wait()
            cp[k][1].wait()

            def add_row(i, carry, bb=bb):
                for j in range(DM // 16):
                    sl = pl.ds(j * 16, 16)
                    r0_v[bb, i, sl] = r0_v[bb, i, sl] + r1_v[bb, i, sl]
                return carry

            lax.fori_loop(0, CC, add_row, 0)
            pltpu.sync_copy(r0_v.at[bb], y_hbm.at[pl.ds(base + k * CC, CC)])

    return combine_kernel(table, s0, s1)


BF = 2048
NFB = DF // BF


def _gelu_new(x):
    return 0.5 * x * (1.0 + jnp.tanh(0.7978845608028654 * (x + 0.044715 * x * x * x)))


def _ffn_body(x_ref, inv_ref, w1_ref, b1_ref, w2_ref, b2_ref, gs_ref,
              out_ref, xb_scr):
    fb = pl.program_id(1)

    @pl.when(fb == 0)
    def _():
        tl = lax.broadcasted_iota(jnp.int32, (CP, T), 1)
        D = (tl == inv_ref[0]).astype(F32)
        xb_scr[...] = jnp.dot(D, x_ref[...], precision=lax.Precision.DEFAULT,
                              preferred_element_type=F32)

    h = jnp.dot(xb_scr[...], w1_ref[0], precision=lax.Precision.DEFAULT,
                preferred_element_type=F32) + b1_ref[0, 0]
    h = _gelu_new(h)
    contrib = jnp.dot(h, w2_ref[0], precision=lax.Precision.DEFAULT,
                      preferred_element_type=F32)

    @pl.when(fb == 0)
    def _():
        out_ref[0] = contrib

    @pl.when(fb > 0)
    def _():
        out_ref[0] = out_ref[0] + contrib

    @pl.when(fb == NFB - 1)
    def _():
        out_ref[0] = (out_ref[0] + b2_ref[0]) * gs_ref[0]


def _ffn_call(x2d, inv, W1, b1, W2, b2, gslot):
    return pl.pallas_call(
        _ffn_body,
        grid=(E, NFB),
        in_specs=[
            pl.BlockSpec((T, DM), lambda e, fb: (0, 0)),
            pl.BlockSpec((1, CP, 1), lambda e, fb: (e, 0, 0)),
            pl.BlockSpec((1, DM, BF), lambda e, fb: (e, 0, fb)),
            pl.BlockSpec((1, 1, 1, BF), lambda e, fb: (e, fb, 0, 0)),
            pl.BlockSpec((1, BF, DM), lambda e, fb: (e, fb, 0)),
            pl.BlockSpec((1, 1, DM), lambda e, fb: (e, 0, 0)),
            pl.BlockSpec((1, CP, 1), lambda e, fb: (e, 0, 0)),
        ],
        out_specs=pl.BlockSpec((1, CP, DM), lambda e, fb: (e, 0, 0)),
        out_shape=jax.ShapeDtypeStruct((E, CP, DM), F32),
        scratch_shapes=[pltpu.VMEM((CP, DM), F32)],
    )(x2d, inv.reshape(E, CP, 1), W1, b1.reshape(E, NFB, 1, BF), W2,
      b2.reshape(E, 1, DM), gslot.reshape(E, CP, 1))


def kernel(hidden_states, Wg, bg, W1, b1, W2, b2, alpha):
    b, s, d = hidden_states.shape
    assert b * s == T and d == DM and Wg.shape == (DM, E)

    x2d = hidden_states.reshape(T, DM)
    inv, slots, gslot = _routing_call(
        x2d, Wg, bg.reshape(1, E), alpha.reshape(1, E))

    out = _ffn_call(x2d, inv, W1, b1, W2, b2, gslot)

    s0 = slots[:, 0].reshape(NW, NC2, CC)
    s1 = slots[:, 1].reshape(NW, NC2, CC)
    y = _combine_call(out.reshape(ROWS, DM), s0, s1)
    return y.reshape(b, s, d)
